# baseline (speedup 1.0000x reference)
import functools

import jax
import jax.numpy as jnp
from jax import lax
from jax.experimental import pallas as pl
from jax.experimental.pallas import tpu as pltpu
from jax.experimental.pallas import tpu_sc as plsc

NC = 2
NS = 16
L = 16
NW = NC * NS

IDX_CHUNK = 128
PACK = 4
TCOLS = 16384


def _cdiv(a, b):
    return -(-a // b)


@functools.lru_cache(maxsize=None)
def _build_tc_relayout(N: int, D: int):
    ROWS = TCOLS // PACK
    nsteps = _cdiv(N, TCOLS)

    def body(u_in, i_in, u_out, i_out):
        eye = jnp.eye(D, dtype=jnp.float32)
        for src, dst in ((u_in, u_out), (i_in, i_out)):
            x = src[...]
            y = lax.dot_general(x, eye, (((0,), (0,)), ((), ())),
                                precision=lax.Precision.HIGHEST,
                                preferred_element_type=jnp.float32)
            dst[...] = jnp.concatenate(
                [lax.slice(y, (k * ROWS, 0), ((k + 1) * ROWS, D))
                 for k in range(PACK)], axis=1)

    in_spec = pl.BlockSpec((D, TCOLS), lambda i: (0, i))
    out_spec = pl.BlockSpec((ROWS, PACK * D), lambda i: (i, 0))
    out_ty = jax.ShapeDtypeStruct((nsteps * ROWS, PACK * D), jnp.float32)
    return pl.pallas_call(
        body,
        grid=(nsteps,),
        in_specs=[in_spec, in_spec],
        out_specs=[out_spec, out_spec],
        out_shape=[out_ty, out_ty],
        compiler_params=pltpu.CompilerParams(
            vmem_limit_bytes=100 * 1024 * 1024),
    )


@functools.lru_cache(maxsize=None)
def _build_sc_kernel(B: int, N: int, D: int):
    assert B % (NW * L) == 0 and D * PACK == 128
    BPW = B // NW
    NIDX = BPW // IDX_CHUNK

    mesh = plsc.VectorSubcoreMesh(core_axis_name="c", subcore_axis_name="s")

    @functools.partial(
        pl.kernel,
        out_type=jax.ShapeDtypeStruct((B // IDX_CHUNK, IDX_CHUNK),
                                      jnp.float32),
        mesh=mesh,
        scratch_types=[
            pltpu.VMEM((NIDX, IDX_CHUNK), jnp.int32),
            pltpu.VMEM((NIDX, IDX_CHUNK), jnp.int32),
            pltpu.VMEM((NIDX, IDX_CHUNK), jnp.int32),
            pltpu.VMEM((NIDX, IDX_CHUNK), jnp.int32),
            pltpu.VMEM((NIDX, IDX_CHUNK), jnp.int32),
            pltpu.VMEM((NIDX, IDX_CHUNK), jnp.int32),
            pltpu.VMEM((IDX_CHUNK, PACK * 32), jnp.float32),
            pltpu.VMEM((IDX_CHUNK, PACK * 32), jnp.float32),
            pltpu.VMEM((IDX_CHUNK, 128), jnp.float32),
            pltpu.VMEM((IDX_CHUNK, 128), jnp.float32),
            pltpu.VMEM((L * L,), jnp.float32),
            pltpu.VMEM((NIDX, IDX_CHUNK), jnp.float32),
            pltpu.SemaphoreType.DMA,
        ],
        compiler_params=pltpu.CompilerParams(needs_layout_passes=False),
    )
    def sc_kernel(uidx_hbm, iidx_hbm, uf_hbm, if_hbm, ub_hbm, ib_hbm,
                  out_hbm, uidx_v, iidx_v, ublk_v, iblk_v, ubb_v, ibb_v,
                  uslab_v, islab_v, ubs_v, ibs_v, prod_v, out_v, sem):
        wid = lax.axis_index("s") * NC + lax.axis_index("c")

        pltpu.sync_copy(uidx_hbm.at[pl.ds(wid * NIDX, NIDX)], uidx_v)
        pltpu.sync_copy(iidx_hbm.at[pl.ds(wid * NIDX, NIDX)], iidx_v)

        iota = lax.iota(jnp.int32, L)

        def blk_body(i, carry):
            j = i // (IDX_CHUNK // L)
            off = (i % (IDX_CHUNK // L)) * L
            s = pl.ds(off, L)
            u = uidx_v[j, s]
            it = iidx_v[j, s]
            rows = TCOLS // PACK
            sb = TCOLS.bit_length() - 1
            ublk_v[j, s] = (u >> sb) * rows + (u & (rows - 1))
            iblk_v[j, s] = (it >> sb) * rows + (it & (rows - 1))
            ubb_v[j, s] = u >> 7
            ibb_v[j, s] = it >> 7
            return carry
        lax.fori_loop(0, BPW // L, blk_body, 0)

        nh = D // L

        for c in range(NIDX):
            copies = [
                pltpu.make_async_copy(uf_hbm.at[ublk_v.at[c]], uslab_v, sem),
                pltpu.make_async_copy(if_hbm.at[iblk_v.at[c]], islab_v, sem),
                pltpu.make_async_copy(ub_hbm.at[ubb_v.at[c]], ubs_v, sem),
                pltpu.make_async_copy(ib_hbm.at[ibb_v.at[c]], ibs_v, sem),
            ]
            for cp in copies:
                cp.start()
            for cp in copies:
                cp.wait()

            def comp_body(e16, carry, c=c):
                ebase = e16 * L
                uvec = uidx_v[c, pl.ds(ebase, L)]
                ivec = iidx_v[c, pl.ds(ebase, L)]
                rb = (TCOLS // PACK).bit_length() - 1
                uoff = ((uvec >> rb) & (PACK - 1)) * D
                ioff = ((ivec >> rb) & (PACK - 1)) * D
                ubcol = uvec & 127
                ibcol = ivec & 127
                for l in range(L):
                    e = ebase + l
                    uo = uoff[l]
                    io = ioff[l]
                    t = (uslab_v[e, pl.ds(uo, L)]
                         * islab_v[e, pl.ds(io, L)])
                    for h in range(1, nh):
                        t += (uslab_v[e, pl.ds(uo + h * L, L)]
                              * islab_v[e, pl.ds(io + h * L, L)])
                    uc = ubcol[l]
                    ic = ibcol[l]
                    ubv = ubs_v[e, pl.ds(uc & 0x70, L)]
                    ibv = ibs_v[e, pl.ds(ic & 0x70, L)]
                    t = t + jnp.where(iota == (uc & 15), ubv, 0.0)
                    t = t + jnp.where(iota == (ic & 15), ibv, 0.0)
                    plsc.store_scatter(prod_v, [iota * L + l], t)
                acc = prod_v[pl.ds(0, L)]
                for d in range(1, L):
                    acc += prod_v[pl.ds(d * L, L)]
                out_v[c, pl.ds(ebase, L)] = acc
                return carry
            lax.fori_loop(0, IDX_CHUNK // L, comp_body, 0)

        pltpu.sync_copy(out_v, out_hbm.at[pl.ds(wid * NIDX, NIDX)])

    return sc_kernel


def kernel(user_item_tuple, user_factors, item_factors, user_biases,
           item_biases):
    uit = user_item_tuple.astype(jnp.int32)
    B = uit.shape[0]
    N, D = user_factors.shape
    u_idx = uit[:, 0].reshape(B // IDX_CHUNK, IDX_CHUNK)
    i_idx = uit[:, 1].reshape(B // IDX_CHUNK, IDX_CHUNK)
    ufr, ifr = _build_tc_relayout(N, D)(user_factors.T, item_factors.T)
    npad = (-N) % 128
    ubr = jnp.pad(user_biases, ((0, npad), (0, 0))).reshape(-1, 128)
    ibr = jnp.pad(item_biases, ((0, npad), (0, 0))).reshape(-1, 128)
    out2 = _build_sc_kernel(B, N, D)(u_idx, i_idx, ufr, ifr, ubr, ibr)
    return out2.reshape(B)


# speedup vs baseline: 1.3072x; 1.3072x over previous
import functools

import jax
import jax.numpy as jnp
from jax import lax
from jax.experimental import pallas as pl
from jax.experimental.pallas import tpu as pltpu
from jax.experimental.pallas import tpu_sc as plsc

NC = 2
NS = 16
L = 16
NW = NC * NS

IDX_CHUNK = 128
PACK = 4
TCOLS = 16384


def _cdiv(a, b):
    return -(-a // b)


@functools.lru_cache(maxsize=None)
def _build_tc_relayout(N: int, D: int):
    ROWS = TCOLS // PACK
    nsteps = _cdiv(N, TCOLS)

    def body(u_in, i_in, u_out, i_out):
        eye = jnp.eye(D, dtype=jnp.float32)
        for src, dst in ((u_in, u_out), (i_in, i_out)):
            x = src[...]
            y = lax.dot_general(x, eye, (((0,), (0,)), ((), ())),
                                precision=lax.Precision.HIGHEST,
                                preferred_element_type=jnp.float32)
            dst[...] = jnp.concatenate(
                [lax.slice(y, (k * ROWS, 0), ((k + 1) * ROWS, D))
                 for k in range(PACK)], axis=1)

    in_spec = pl.BlockSpec((D, TCOLS), lambda i: (0, i))
    out_spec = pl.BlockSpec((ROWS, PACK * D), lambda i: (i, 0))
    out_ty = jax.ShapeDtypeStruct((nsteps * ROWS, PACK * D), jnp.float32)
    return pl.pallas_call(
        body,
        grid=(nsteps,),
        in_specs=[in_spec, in_spec],
        out_specs=[out_spec, out_spec],
        out_shape=[out_ty, out_ty],
        compiler_params=pltpu.CompilerParams(
            vmem_limit_bytes=100 * 1024 * 1024),
    )


@functools.lru_cache(maxsize=None)
def _build_sc_kernel(B: int, N: int, D: int):
    assert B % (NW * L) == 0 and D * PACK == 128
    BPW = B // NW
    NIDX = BPW // IDX_CHUNK

    mesh = plsc.VectorSubcoreMesh(core_axis_name="c", subcore_axis_name="s")

    @functools.partial(
        pl.kernel,
        out_type=jax.ShapeDtypeStruct((B // IDX_CHUNK, IDX_CHUNK),
                                      jnp.float32),
        mesh=mesh,
        scratch_types=[
            pltpu.VMEM((NIDX, IDX_CHUNK), jnp.int32),
            pltpu.VMEM((NIDX, IDX_CHUNK), jnp.int32),
            pltpu.VMEM((NIDX, IDX_CHUNK), jnp.int32),
            pltpu.VMEM((NIDX, IDX_CHUNK), jnp.int32),
            pltpu.VMEM((NIDX, IDX_CHUNK), jnp.int32),
            pltpu.VMEM((NIDX, IDX_CHUNK), jnp.int32),
            pltpu.VMEM((IDX_CHUNK, PACK * 32), jnp.float32),
            pltpu.VMEM((IDX_CHUNK, PACK * 32), jnp.float32),
            pltpu.VMEM((IDX_CHUNK, 128), jnp.float32),
            pltpu.VMEM((IDX_CHUNK, 128), jnp.float32),
            pltpu.VMEM((L * L,), jnp.float32),
            pltpu.VMEM((NIDX, IDX_CHUNK), jnp.float32),
            pltpu.SemaphoreType.DMA,
        ],
        compiler_params=pltpu.CompilerParams(needs_layout_passes=False),
    )
    def sc_kernel(uidx_hbm, iidx_hbm, uf_hbm, if_hbm, ub_hbm, ib_hbm,
                  out_hbm, uidx_v, iidx_v, ublk_v, iblk_v, ubb_v, ibb_v,
                  uslab_v, islab_v, ubs_v, ibs_v, prod_v, out_v, sem):
        wid = lax.axis_index("s") * NC + lax.axis_index("c")

        pltpu.sync_copy(uidx_hbm.at[pl.ds(wid * NIDX, NIDX)], uidx_v)
        pltpu.sync_copy(iidx_hbm.at[pl.ds(wid * NIDX, NIDX)], iidx_v)

        iota = lax.iota(jnp.int32, L)

        def blk_body(i, carry):
            j = i // (IDX_CHUNK // L)
            off = (i % (IDX_CHUNK // L)) * L
            s = pl.ds(off, L)
            u = uidx_v[j, s]
            it = iidx_v[j, s]
            ublk_v[j, s] = u >> 2
            iblk_v[j, s] = it >> 2
            ubb_v[j, s] = u >> 7
            ibb_v[j, s] = it >> 7
            return carry
        lax.fori_loop(0, BPW // L, blk_body, 0)

        nh = D // L

        for c in range(NIDX):
            copies = [
                pltpu.make_async_copy(uf_hbm.at[ublk_v.at[c]], uslab_v, sem),
                pltpu.make_async_copy(if_hbm.at[iblk_v.at[c]], islab_v, sem),
                pltpu.make_async_copy(ub_hbm.at[ubb_v.at[c]], ubs_v, sem),
                pltpu.make_async_copy(ib_hbm.at[ibb_v.at[c]], ibs_v, sem),
            ]
            for cp in copies:
                cp.start()
            for cp in copies:
                cp.wait()

            def comp_body(e16, carry, c=c):
                ebase = e16 * L
                uvec = uidx_v[c, pl.ds(ebase, L)]
                ivec = iidx_v[c, pl.ds(ebase, L)]
                uoff = (uvec & (PACK - 1)) * D
                ioff = (ivec & (PACK - 1)) * D
                ubcol = uvec & 127
                ibcol = ivec & 127
                for l in range(L):
                    e = ebase + l
                    uo = uoff[l]
                    io = ioff[l]
                    t = (uslab_v[e, pl.ds(uo, L)]
                         * islab_v[e, pl.ds(io, L)])
                    for h in range(1, nh):
                        t += (uslab_v[e, pl.ds(uo + h * L, L)]
                              * islab_v[e, pl.ds(io + h * L, L)])
                    uc = ubcol[l]
                    ic = ibcol[l]
                    ubv = ubs_v[e, pl.ds(uc & 0x70, L)]
                    ibv = ibs_v[e, pl.ds(ic & 0x70, L)]
                    t = t + jnp.where(iota == (uc & 15), ubv, 0.0)
                    t = t + jnp.where(iota == (ic & 15), ibv, 0.0)
                    plsc.store_scatter(prod_v, [iota * L + l], t)
                acc = prod_v[pl.ds(0, L)]
                for d in range(1, L):
                    acc += prod_v[pl.ds(d * L, L)]
                out_v[c, pl.ds(ebase, L)] = acc
                return carry
            lax.fori_loop(0, IDX_CHUNK // L, comp_body, 0)

        pltpu.sync_copy(out_v, out_hbm.at[pl.ds(wid * NIDX, NIDX)])

    return sc_kernel


def kernel(user_item_tuple, user_factors, item_factors, user_biases,
           item_biases):
    uit = user_item_tuple.astype(jnp.int32)
    B = uit.shape[0]
    N, D = user_factors.shape
    u_idx = uit[:, 0].reshape(B // IDX_CHUNK, IDX_CHUNK)
    i_idx = uit[:, 1].reshape(B // IDX_CHUNK, IDX_CHUNK)
    ufr = user_factors.reshape(N // PACK, PACK * D)
    ifr = item_factors.reshape(N // PACK, PACK * D)
    npad = (-N) % 128
    ubr = jnp.pad(user_biases, ((0, npad), (0, 0))).reshape(-1, 128)
    ibr = jnp.pad(item_biases, ((0, npad), (0, 0))).reshape(-1, 128)
    out2 = _build_sc_kernel(B, N, D)(u_idx, i_idx, ufr, ifr, ubr, ibr)
    return out2.reshape(B)


# speedup vs baseline: 1.9566x; 1.4968x over previous
import functools

import jax
import jax.numpy as jnp
from jax import lax
from jax.experimental import pallas as pl
from jax.experimental.pallas import tpu as pltpu
from jax.experimental.pallas import tpu_sc as plsc

NC = 2
NS = 16
L = 16
NW = NC * NS

IDX_CHUNK = 128
PACK = 4
TCOLS = 8192


def _cdiv(a, b):
    return -(-a // b)


@functools.lru_cache(maxsize=None)
def _build_tc_relayout(N: int, D: int):
    ROWS = TCOLS // PACK
    nsteps = _cdiv(N, TCOLS)

    def body(u_in, i_in, u_out, i_out):
        for src, dst in ((u_in, u_out), (i_in, i_out)):
            y = src[...].T
            dst[...] = jnp.concatenate(
                [lax.slice(y, (k * ROWS, 0), ((k + 1) * ROWS, D))
                 for k in range(PACK)], axis=1)

    in_spec = pl.BlockSpec((D, TCOLS), lambda i: (0, i))
    out_spec = pl.BlockSpec((ROWS, PACK * D), lambda i: (i, 0))
    out_ty = jax.ShapeDtypeStruct((nsteps * ROWS, PACK * D), jnp.float32)
    return pl.pallas_call(
        body,
        grid=(nsteps,),
        in_specs=[in_spec, in_spec],
        out_specs=[out_spec, out_spec],
        out_shape=[out_ty, out_ty],
        compiler_params=pltpu.CompilerParams(
            vmem_limit_bytes=100 * 1024 * 1024),
    )


@functools.lru_cache(maxsize=None)
def _build_sc_kernel(B: int, N: int, D: int):
    assert B % (NW * L) == 0 and D * PACK == 128
    BPW = B // NW
    NIDX = BPW // IDX_CHUNK

    mesh = plsc.VectorSubcoreMesh(core_axis_name="c", subcore_axis_name="s")

    @functools.partial(
        pl.kernel,
        out_type=jax.ShapeDtypeStruct((B // IDX_CHUNK, IDX_CHUNK),
                                      jnp.float32),
        mesh=mesh,
        scratch_types=[
            pltpu.VMEM((NIDX, IDX_CHUNK), jnp.int32),
            pltpu.VMEM((NIDX, IDX_CHUNK), jnp.int32),
            pltpu.VMEM((NIDX, IDX_CHUNK), jnp.int32),
            pltpu.VMEM((NIDX, IDX_CHUNK), jnp.int32),
            pltpu.VMEM((NIDX, IDX_CHUNK), jnp.int32),
            pltpu.VMEM((NIDX, IDX_CHUNK), jnp.int32),
            pltpu.VMEM((IDX_CHUNK, PACK * 32), jnp.float32),
            pltpu.VMEM((IDX_CHUNK, PACK * 32), jnp.float32),
            pltpu.VMEM((IDX_CHUNK, 128), jnp.float32),
            pltpu.VMEM((IDX_CHUNK, 128), jnp.float32),
            pltpu.VMEM((L * L,), jnp.float32),
            pltpu.VMEM((NIDX, IDX_CHUNK), jnp.float32),
            pltpu.SemaphoreType.DMA,
        ],
        compiler_params=pltpu.CompilerParams(needs_layout_passes=False),
    )
    def sc_kernel(uidx_hbm, iidx_hbm, uf_hbm, if_hbm, ub_hbm, ib_hbm,
                  out_hbm, uidx_v, iidx_v, ublk_v, iblk_v, ubb_v, ibb_v,
                  uslab_v, islab_v, ubs_v, ibs_v, prod_v, out_v, sem):
        wid = lax.axis_index("s") * NC + lax.axis_index("c")

        pltpu.sync_copy(uidx_hbm.at[pl.ds(wid * NIDX, NIDX)], uidx_v)
        pltpu.sync_copy(iidx_hbm.at[pl.ds(wid * NIDX, NIDX)], iidx_v)

        iota = lax.iota(jnp.int32, L)

        def blk_body(i, carry):
            j = i // (IDX_CHUNK // L)
            off = (i % (IDX_CHUNK // L)) * L
            s = pl.ds(off, L)
            u = uidx_v[j, s]
            it = iidx_v[j, s]
            rows = TCOLS // PACK
            sb = TCOLS.bit_length() - 1
            ublk_v[j, s] = (u >> sb) * rows + (u & (rows - 1))
            iblk_v[j, s] = (it >> sb) * rows + (it & (rows - 1))
            ubb_v[j, s] = u >> 7
            ibb_v[j, s] = it >> 7
            return carry
        lax.fori_loop(0, BPW // L, blk_body, 0)

        nh = D // L

        for c in range(NIDX):
            copies = [
                pltpu.make_async_copy(uf_hbm.at[ublk_v.at[c]], uslab_v, sem),
                pltpu.make_async_copy(if_hbm.at[iblk_v.at[c]], islab_v, sem),
                pltpu.make_async_copy(ub_hbm.at[ubb_v.at[c]], ubs_v, sem),
                pltpu.make_async_copy(ib_hbm.at[ibb_v.at[c]], ibs_v, sem),
            ]
            for cp in copies:
                cp.start()
            for cp in copies:
                cp.wait()

            def comp_body(e16, carry, c=c):
                ebase = e16 * L
                uvec = uidx_v[c, pl.ds(ebase, L)]
                ivec = iidx_v[c, pl.ds(ebase, L)]
                rb = (TCOLS // PACK).bit_length() - 1
                uoff = ((uvec >> rb) & (PACK - 1)) * D
                ioff = ((ivec >> rb) & (PACK - 1)) * D
                ubcol = uvec & 127
                ibcol = ivec & 127
                for l in range(L):
                    e = ebase + l
                    uo = uoff[l]
                    io = ioff[l]
                    t = (uslab_v[e, pl.ds(uo, L)]
                         * islab_v[e, pl.ds(io, L)])
                    for h in range(1, nh):
                        t += (uslab_v[e, pl.ds(uo + h * L, L)]
                              * islab_v[e, pl.ds(io + h * L, L)])
                    uc = ubcol[l]
                    ic = ibcol[l]
                    ubv = ubs_v[e, pl.ds(uc & 0x70, L)]
                    ibv = ibs_v[e, pl.ds(ic & 0x70, L)]
                    t = t + jnp.where(iota == (uc & 15), ubv, 0.0)
                    t = t + jnp.where(iota == (ic & 15), ibv, 0.0)
                    plsc.store_scatter(prod_v, [iota * L + l], t)
                acc = prod_v[pl.ds(0, L)]
                for d in range(1, L):
                    acc += prod_v[pl.ds(d * L, L)]
                out_v[c, pl.ds(ebase, L)] = acc
                return carry
            lax.fori_loop(0, IDX_CHUNK // L, comp_body, 0)

        pltpu.sync_copy(out_v, out_hbm.at[pl.ds(wid * NIDX, NIDX)])

    return sc_kernel


def kernel(user_item_tuple, user_factors, item_factors, user_biases,
           item_biases):
    uit = user_item_tuple.astype(jnp.int32)
    B = uit.shape[0]
    N, D = user_factors.shape
    u_idx = uit[:, 0].reshape(B // IDX_CHUNK, IDX_CHUNK)
    i_idx = uit[:, 1].reshape(B // IDX_CHUNK, IDX_CHUNK)
    ufr, ifr = _build_tc_relayout(N, D)(user_factors.T, item_factors.T)
    npad = (-N) % 128
    ubr = jnp.pad(user_biases, ((0, npad), (0, 0))).reshape(-1, 128)
    ibr = jnp.pad(item_biases, ((0, npad), (0, 0))).reshape(-1, 128)
    out2 = _build_sc_kernel(B, N, D)(u_idx, i_idx, ufr, ifr, ubr, ibr)
    return out2.reshape(B)


# speedup vs baseline: 1.9624x; 1.0030x over previous
import functools

import jax
import jax.numpy as jnp
from jax import lax
from jax.experimental import pallas as pl
from jax.experimental.pallas import tpu as pltpu
from jax.experimental.pallas import tpu_sc as plsc

NC = 2
NS = 16
L = 16
NW = NC * NS

IDX_CHUNK = 128
PACK = 4
TCOLS = 8192


def _cdiv(a, b):
    return -(-a // b)


@functools.lru_cache(maxsize=None)
def _build_tc_relayout(N: int, D: int):
    ROWS = TCOLS // PACK
    nsteps = _cdiv(N, TCOLS)

    def body(u_in, i_in, u_out, i_out):
        for src, dst in ((u_in, u_out), (i_in, i_out)):
            for k in range(PACK):
                y = src[:, pl.ds(k * ROWS, ROWS)].T
                dst[:, pl.ds(k * D, D)] = y

    in_spec = pl.BlockSpec((D, TCOLS), lambda i: (0, i))
    out_spec = pl.BlockSpec((ROWS, PACK * D), lambda i: (i, 0))
    out_ty = jax.ShapeDtypeStruct((nsteps * ROWS, PACK * D), jnp.float32)
    return pl.pallas_call(
        body,
        grid=(nsteps,),
        in_specs=[in_spec, in_spec],
        out_specs=[out_spec, out_spec],
        out_shape=[out_ty, out_ty],
        compiler_params=pltpu.CompilerParams(
            vmem_limit_bytes=100 * 1024 * 1024),
    )


@functools.lru_cache(maxsize=None)
def _build_sc_kernel(B: int, N: int, D: int):
    assert B % (NW * L) == 0 and D * PACK == 128
    BPW = B // NW
    NIDX = BPW // IDX_CHUNK

    mesh = plsc.VectorSubcoreMesh(core_axis_name="c", subcore_axis_name="s")

    @functools.partial(
        pl.kernel,
        out_type=jax.ShapeDtypeStruct((B // IDX_CHUNK, IDX_CHUNK),
                                      jnp.float32),
        mesh=mesh,
        scratch_types=[
            pltpu.VMEM((NIDX, IDX_CHUNK), jnp.int32),
            pltpu.VMEM((NIDX, IDX_CHUNK), jnp.int32),
            pltpu.VMEM((NIDX, IDX_CHUNK), jnp.int32),
            pltpu.VMEM((NIDX, IDX_CHUNK), jnp.int32),
            pltpu.VMEM((NIDX, IDX_CHUNK), jnp.int32),
            pltpu.VMEM((NIDX, IDX_CHUNK), jnp.int32),
            pltpu.VMEM((IDX_CHUNK, PACK * 32), jnp.float32),
            pltpu.VMEM((IDX_CHUNK, PACK * 32), jnp.float32),
            pltpu.VMEM((IDX_CHUNK, 128), jnp.float32),
            pltpu.VMEM((IDX_CHUNK, 128), jnp.float32),
            pltpu.VMEM((L * L,), jnp.float32),
            pltpu.VMEM((NIDX, IDX_CHUNK), jnp.float32),
            pltpu.SemaphoreType.DMA,
        ],
        compiler_params=pltpu.CompilerParams(needs_layout_passes=False),
    )
    def sc_kernel(uidx_hbm, iidx_hbm, uf_hbm, if_hbm, ub_hbm, ib_hbm,
                  out_hbm, uidx_v, iidx_v, ublk_v, iblk_v, ubb_v, ibb_v,
                  uslab_v, islab_v, ubs_v, ibs_v, prod_v, out_v, sem):
        wid = lax.axis_index("s") * NC + lax.axis_index("c")

        pltpu.sync_copy(uidx_hbm.at[pl.ds(wid * NIDX, NIDX)], uidx_v)
        pltpu.sync_copy(iidx_hbm.at[pl.ds(wid * NIDX, NIDX)], iidx_v)

        iota = lax.iota(jnp.int32, L)

        def blk_body(i, carry):
            j = i // (IDX_CHUNK // L)
            off = (i % (IDX_CHUNK // L)) * L
            s = pl.ds(off, L)
            u = uidx_v[j, s]
            it = iidx_v[j, s]
            rows = TCOLS // PACK
            sb = TCOLS.bit_length() - 1
            ublk_v[j, s] = (u >> sb) * rows + (u & (rows - 1))
            iblk_v[j, s] = (it >> sb) * rows + (it & (rows - 1))
            ubb_v[j, s] = u >> 7
            ibb_v[j, s] = it >> 7
            return carry
        lax.fori_loop(0, BPW // L, blk_body, 0)

        nh = D // L

        for c in range(NIDX):
            copies = [
                pltpu.make_async_copy(uf_hbm.at[ublk_v.at[c]], uslab_v, sem),
                pltpu.make_async_copy(if_hbm.at[iblk_v.at[c]], islab_v, sem),
                pltpu.make_async_copy(ub_hbm.at[ubb_v.at[c]], ubs_v, sem),
                pltpu.make_async_copy(ib_hbm.at[ibb_v.at[c]], ibs_v, sem),
            ]
            for cp in copies:
                cp.start()
            for cp in copies:
                cp.wait()

            def comp_body(e16, carry, c=c):
                ebase = e16 * L
                uvec = uidx_v[c, pl.ds(ebase, L)]
                ivec = iidx_v[c, pl.ds(ebase, L)]
                rb = (TCOLS // PACK).bit_length() - 1
                uoff = ((uvec >> rb) & (PACK - 1)) * D
                ioff = ((ivec >> rb) & (PACK - 1)) * D
                ubcol = uvec & 127
                ibcol = ivec & 127
                for l in range(L):
                    e = ebase + l
                    uo = uoff[l]
                    io = ioff[l]
                    t = (uslab_v[e, pl.ds(uo, L)]
                         * islab_v[e, pl.ds(io, L)])
                    for h in range(1, nh):
                        t += (uslab_v[e, pl.ds(uo + h * L, L)]
                              * islab_v[e, pl.ds(io + h * L, L)])
                    uc = ubcol[l]
                    ic = ibcol[l]
                    ubv = ubs_v[e, pl.ds(uc & 0x70, L)]
                    ibv = ibs_v[e, pl.ds(ic & 0x70, L)]
                    t = t + jnp.where(iota == (uc & 15), ubv, 0.0)
                    t = t + jnp.where(iota == (ic & 15), ibv, 0.0)
                    plsc.store_scatter(prod_v, [iota * L + l], t)
                acc = prod_v[pl.ds(0, L)]
                for d in range(1, L):
                    acc += prod_v[pl.ds(d * L, L)]
                out_v[c, pl.ds(ebase, L)] = acc
                return carry
            lax.fori_loop(0, IDX_CHUNK // L, comp_body, 0)

        pltpu.sync_copy(out_v, out_hbm.at[pl.ds(wid * NIDX, NIDX)])

    return sc_kernel


def kernel(user_item_tuple, user_factors, item_factors, user_biases,
           item_biases):
    uit = user_item_tuple.astype(jnp.int32)
    B = uit.shape[0]
    N, D = user_factors.shape
    u_idx = uit[:, 0].reshape(B // IDX_CHUNK, IDX_CHUNK)
    i_idx = uit[:, 1].reshape(B // IDX_CHUNK, IDX_CHUNK)
    ufr, ifr = _build_tc_relayout(N, D)(user_factors.T, item_factors.T)
    npad = (-N) % 128
    ubr = jnp.pad(user_biases, ((0, npad), (0, 0))).reshape(-1, 128)
    ibr = jnp.pad(item_biases, ((0, npad), (0, 0))).reshape(-1, 128)
    out2 = _build_sc_kernel(B, N, D)(u_idx, i_idx, ufr, ifr, ubr, ibr)
    return out2.reshape(B)
